# Initial kernel scaffold; baseline (speedup 1.0000x reference)
import jax
import jax.numpy as jnp
from jax.experimental import pallas as pl


def kernel(x, edge_index, W0a, b0a, W0b, b0b, W1a, b1a, W1b, b1b, W2a, b2a, W2b, b2b):
    raise NotImplementedError("write your pallas kernel here")


# speedup vs baseline: 8.7574x; 8.7574x over previous
import functools

import jax
import jax.numpy as jnp
from jax import lax
from jax.experimental import pallas as pl
from jax.experimental.pallas import tpu as pltpu
from jax.experimental.pallas import tpu_sc as plsc

N = 10000
E = 320000
NP = 10240
RPT = NP // 16
CB = 128
NCH_W = 160
NCH_1 = 80
GRP = 16
EP = 16 * NCH_W * CB


def _make_prop_wide(dh):
    mesh = plsc.VectorSubcoreMesh(core_axis_name="c", subcore_axis_name="s")

    @functools.partial(
        pl.kernel,
        mesh=mesh,
        out_type=jax.ShapeDtypeStruct((2, NP, dh), jnp.float32),
        compiler_params=pltpu.CompilerParams(use_tc_tiling_on_sc=False),
        scratch_types=[
            pltpu.VMEM_SHARED((NP, dh), jnp.float32),
            pltpu.VMEM((GRP, CB), jnp.int32),
            pltpu.VMEM((GRP, CB), jnp.int32),
            pltpu.VMEM((CB, dh), jnp.float32),
            pltpu.SemaphoreType.DMA,
        ],
    )
    def k(tab, srcs, dsts, out, acc, src_m, dst_m, rows_v, sem):
        c = lax.axis_index("c")
        s = lax.axis_index("s")
        r0 = s * RPT
        pltpu.sync_copy(tab.at[pl.ds(c * NP + r0, RPT)], acc.at[pl.ds(r0, RPT)])
        plsc.subcore_barrier()

        def outer(g, carry):
            pltpu.sync_copy(srcs.at[c, s, pl.ds(g * GRP, GRP)], src_m)
            pltpu.sync_copy(dsts.at[s, pl.ds(g * GRP, GRP)], dst_m)

            def body(kk, carry2):
                pltpu.async_copy(tab.at[src_m.at[kk]], rows_v, sem).wait()
                pltpu.sync_copy(rows_v, acc.at[dst_m.at[kk]], add=True)
                return carry2

            lax.fori_loop(0, GRP, body, 0)
            return carry

        lax.fori_loop(0, NCH_W // GRP, outer, 0)
        plsc.subcore_barrier()
        pltpu.sync_copy(acc.at[pl.ds(r0, RPT)], out.at[c, pl.ds(r0, RPT)])

    return k


_prop32 = _make_prop_wide(32)
_prop128 = _make_prop_wide(128)


def _make_prop_d1():
    mesh = plsc.VectorSubcoreMesh(core_axis_name="c", subcore_axis_name="s")

    @functools.partial(
        pl.kernel,
        mesh=mesh,
        out_type=jax.ShapeDtypeStruct((2, NP), jnp.float32),
        compiler_params=pltpu.CompilerParams(use_tc_tiling_on_sc=False),
        scratch_types=[
            pltpu.VMEM_SHARED((NP,), jnp.float32),
            pltpu.VMEM((NCH_1, CB), jnp.int32),
            pltpu.VMEM((NCH_1, CB), jnp.int32),
            pltpu.VMEM((CB,), jnp.float32),
            pltpu.VMEM((RPT,), jnp.float32),
            pltpu.SemaphoreType.DMA,
        ],
    )
    def k(tab, srcs, dsts, out, acc, src_m, dst_m, vals_v, zbuf, sem):
        c = lax.axis_index("c")
        s = lax.axis_index("s")
        w = 2 * s + c
        for i in range(RPT // 16):
            zbuf[pl.ds(i * 16, 16)] = jnp.zeros((16,), jnp.float32)
        pltpu.sync_copy(zbuf, acc.at[pl.ds(s * RPT, RPT)])
        pltpu.sync_copy(srcs.at[w], src_m)
        pltpu.sync_copy(dsts.at[w], dst_m)
        plsc.subcore_barrier()

        def body(kk, carry):
            pltpu.async_copy(tab.at[src_m.at[kk]], vals_v, sem).wait()
            pltpu.sync_copy(vals_v, acc.at[dst_m.at[kk]], add=True)
            return carry

        lax.fori_loop(0, NCH_1, body, 0)
        plsc.subcore_barrier()
        pltpu.sync_copy(acc.at[pl.ds(s * RPT, RPT)], out.at[c, pl.ds(s * RPT, RPT)])

    return k


_prop_d1 = _make_prop_d1()


def _pad_tab(tab_ref, u, dh):
    tab_ref[pl.ds(0, N)] = u[:, :dh]
    tab_ref[pl.ds(NP, N)] = u[:, dh:]
    zpad = jnp.zeros((NP - N, dh), jnp.float32)
    tab_ref[pl.ds(N, NP - N)] = zpad
    tab_ref[pl.ds(NP + N, NP - N)] = zpad


def _merge(s_ref):
    v = s_ref[...]
    return jnp.concatenate([v[0, :N], v[1, :N]], axis=1)


def _t1(x_ref, w_ref, cnt_ref, dis_ref, tab_ref):
    deg = 1.0 + cnt_ref[0] + cnt_ref[1]
    dis = lax.rsqrt(deg)
    dis_ref[...] = dis
    z = jnp.dot(x_ref[...], w_ref[...], preferred_element_type=jnp.float32)
    u = z * dis[:N]
    _pad_tab(tab_ref, u, 32)


def _t2(s_ref, dis_ref, b_ref, w_ref, tab_ref):
    dis = dis_ref[...]
    h = jnp.maximum(dis[:N] * _merge(s_ref) + b_ref[...], 0.0)
    u = dis[:N] * jnp.dot(h, w_ref[...], preferred_element_type=jnp.float32)
    _pad_tab(tab_ref, u, 32)


def _t3(s_ref, dis_ref, b_ref, tab_ref):
    dis = dis_ref[...]
    p = dis[:N] * _merge(s_ref) + b_ref[...]
    h = jnp.where(p > 0, p, 0.1 * p)
    _pad_tab(tab_ref, dis[:N] * h, 32)


def _t4(s_ref, dis_ref, ba_ref, wa_ref, wb_ref, tab_ref):
    dis = dis_ref[...]
    p = dis[:N] * _merge(s_ref)
    h = jnp.maximum(jnp.dot(p, wa_ref[...], preferred_element_type=jnp.float32)
                    + ba_ref[...], 0.0)
    u = dis[:N] * jnp.dot(h, wb_ref[...], preferred_element_type=jnp.float32)
    _pad_tab(tab_ref, u, 128)


def _t5(s_ref, dis_ref, b_ref, tab_ref):
    dis = dis_ref[...]
    p = dis[:N] * _merge(s_ref) + b_ref[...]
    h = jnp.where(p > 0, p, 0.1 * p)
    _pad_tab(tab_ref, dis[:N] * h, 128)


def _t6(s_ref, dis_ref, ba_ref, wa_ref, wb_ref, tab_ref):
    dis = dis_ref[...]
    p = dis[:N] * _merge(s_ref)
    h = jnp.maximum(jnp.dot(p, wa_ref[...], preferred_element_type=jnp.float32)
                    + ba_ref[...], 0.0)
    z = jnp.sum(h * wb_ref[...], axis=1, keepdims=True)
    tab_ref[pl.ds(0, N)] = dis[:N] * z
    tab_ref[pl.ds(N, NP - N)] = jnp.zeros((NP - N, 1), jnp.float32)


def _t7(t6_ref, u6_ref, dis_ref, b_ref, out_ref):
    v = t6_ref[...]
    out_ref[...] = (dis_ref[pl.ds(0, N)]
                    * (u6_ref[pl.ds(0, N)] + v[0, :N] + v[1, :N]) + b_ref[...])


def _tc(body, out_shape, *args):
    return pl.pallas_call(body, out_shape=out_shape)(*args)


def kernel(x, edge_index, W0a, b0a, W0b, b0b, W1a, b1a, W1b, b1b, W2a, b2a, W2b, b2b):
    f32 = jnp.float32
    src = edge_index[0]
    dst = edge_index[1]
    srcp = jnp.concatenate([src, jnp.full((EP - E,), N, jnp.int32)])
    dstp = jnp.concatenate([dst, jnp.full((EP - E,), N, jnp.int32)])
    offs = jnp.array([0, NP], jnp.int32)
    srcs_w = (srcp[None, :] + offs[:, None]).reshape(2, 16, NCH_W, CB)
    dst_w = dstp.reshape(16, NCH_W, CB)
    src_1 = srcp.reshape(32, NCH_1, CB)
    dst_1 = dstp.reshape(32, NCH_1, CB)

    b0a_ = b0a.reshape(1, -1)
    b0b_ = b0b.reshape(1, -1)
    b1a_ = b1a.reshape(1, -1)
    b1b_ = b1b.reshape(1, -1)
    b2a_ = b2a.reshape(1, -1)
    b2b_ = b2b.reshape(1, -1)
    w2b_row = W2b.reshape(1, -1)

    ones_tab = jnp.zeros((NP,), f32).at[:N].set(1.0)
    cnt = _prop_d1(ones_tab, src_1, dst_1)
    cnt3 = cnt.reshape(2, NP, 1)

    sd = jax.ShapeDtypeStruct
    dis, tab1 = _tc(_t1, [sd((NP, 1), f32), sd((2 * NP, 32), f32)],
                    x, W0a, cnt3)
    s1 = _prop32(tab1, srcs_w, dst_w)
    tab2 = _tc(_t2, sd((2 * NP, 32), f32), s1, dis, b0a_, W0b)
    s2 = _prop32(tab2, srcs_w, dst_w)
    tab3 = _tc(_t3, sd((2 * NP, 32), f32), s2, dis, b0b_)
    s3 = _prop32(tab3, srcs_w, dst_w)
    tab4 = _tc(_t4, sd((2 * NP, 128), f32), s3, dis, b1a_, W1a, W1b)
    s4 = _prop128(tab4, srcs_w, dst_w)
    tab5 = _tc(_t5, sd((2 * NP, 128), f32), s4, dis, b1b_)
    s5 = _prop128(tab5, srcs_w, dst_w)
    tab6 = _tc(_t6, sd((NP, 1), f32), s5, dis, b2a_, W2a, w2b_row)
    t6 = _prop_d1(tab6.reshape(NP), src_1, dst_1)
    out = _tc(_t7, sd((N, 1), f32), t6.reshape(2, NP, 1), tab6, dis, b2b_)
    return out


# speedup vs baseline: 11.2006x; 1.2790x over previous
import functools

import jax
import jax.numpy as jnp
from jax import lax
from jax.experimental import pallas as pl
from jax.experimental.pallas import tpu as pltpu
from jax.experimental.pallas import tpu_sc as plsc

N = 10000
E = 320000
NP = 10240
RPT = NP // 16
CB = 128
NCH_W = 160
NCH_1 = 80
GRP = 16
EP = 16 * NCH_W * CB


def _make_prop_wide(dh, nb):
    mesh = plsc.VectorSubcoreMesh(core_axis_name="c", subcore_axis_name="s")

    @functools.partial(
        pl.kernel,
        mesh=mesh,
        out_type=jax.ShapeDtypeStruct((2, NP, dh), jnp.float32),
        compiler_params=pltpu.CompilerParams(use_tc_tiling_on_sc=False),
        scratch_types=[
            pltpu.VMEM_SHARED((NP, dh), jnp.float32),
            pltpu.VMEM((GRP, CB), jnp.int32),
            pltpu.VMEM((GRP, CB), jnp.int32),
            pltpu.VMEM((nb, CB, dh), jnp.float32),
            pltpu.SemaphoreType.DMA((nb,)),
            pltpu.SemaphoreType.DMA((nb,)),
        ],
    )
    def k(tab, srcs, dsts, out, acc, src_m, dst_m, bufs, gsem, ssem):
        c = lax.axis_index("c")
        s = lax.axis_index("s")
        r0 = s * RPT
        pltpu.sync_copy(tab.at[pl.ds(c * NP + r0, RPT)], acc.at[pl.ds(r0, RPT)])
        plsc.subcore_barrier()

        def outer(g, carry):
            pltpu.sync_copy(srcs.at[c, s, pl.ds(g * GRP, GRP)], src_m)
            pltpu.sync_copy(dsts.at[s, pl.ds(g * GRP, GRP)], dst_m)
            gh = [None] * nb
            sh = [None] * nb
            for b in range(nb):
                gh[b] = pltpu.async_copy(tab.at[src_m.at[b]], bufs.at[b],
                                         gsem.at[b])
            for kk in range(GRP):
                b = kk % nb
                gh[b].wait()
                sh[b] = pltpu.async_copy(bufs.at[b], acc.at[dst_m.at[kk]],
                                         ssem.at[b], add=True)
                nk = kk + nb
                if nk < GRP:
                    sh[b].wait()
                    gh[b] = pltpu.async_copy(tab.at[src_m.at[nk]], bufs.at[b],
                                             gsem.at[b])
            for kk in range(GRP - nb, GRP):
                sh[kk % nb].wait()
            return carry

        lax.fori_loop(0, NCH_W // GRP, outer, 0)
        plsc.subcore_barrier()
        pltpu.sync_copy(acc.at[pl.ds(r0, RPT)], out.at[c, pl.ds(r0, RPT)])

    return k


_prop32 = _make_prop_wide(32, 4)
_prop128 = _make_prop_wide(128, 2)


def _make_prop_d1():
    mesh = plsc.VectorSubcoreMesh(core_axis_name="c", subcore_axis_name="s")

    @functools.partial(
        pl.kernel,
        mesh=mesh,
        out_type=jax.ShapeDtypeStruct((2, NP), jnp.float32),
        compiler_params=pltpu.CompilerParams(use_tc_tiling_on_sc=False),
        scratch_types=[
            pltpu.VMEM_SHARED((NP,), jnp.float32),
            pltpu.VMEM((NCH_1, CB), jnp.int32),
            pltpu.VMEM((NCH_1, CB), jnp.int32),
            pltpu.VMEM((4, CB), jnp.float32),
            pltpu.VMEM((RPT,), jnp.float32),
            pltpu.SemaphoreType.DMA((4,)),
            pltpu.SemaphoreType.DMA((4,)),
        ],
    )
    def k(tab, srcs, dsts, out, acc, src_m, dst_m, bufs, zbuf, gsem, ssem):
        nb = 4
        c = lax.axis_index("c")
        s = lax.axis_index("s")
        w = 2 * s + c
        for i in range(RPT // 16):
            zbuf[pl.ds(i * 16, 16)] = jnp.zeros((16,), jnp.float32)
        pltpu.sync_copy(zbuf, acc.at[pl.ds(s * RPT, RPT)])
        pltpu.sync_copy(srcs.at[w], src_m)
        pltpu.sync_copy(dsts.at[w], dst_m)
        plsc.subcore_barrier()

        def outer(g, carry):
            g0 = g * GRP
            gh = [None] * nb
            sh = [None] * nb
            for b in range(nb):
                gh[b] = pltpu.async_copy(tab.at[src_m.at[g0 + b]], bufs.at[b],
                                         gsem.at[b])
            for kk in range(GRP):
                b = kk % nb
                gh[b].wait()
                sh[b] = pltpu.async_copy(bufs.at[b], acc.at[dst_m.at[g0 + kk]],
                                         ssem.at[b], add=True)
                nk = kk + nb
                if nk < GRP:
                    sh[b].wait()
                    gh[b] = pltpu.async_copy(tab.at[src_m.at[g0 + nk]],
                                             bufs.at[b], gsem.at[b])
            for kk in range(GRP - nb, GRP):
                sh[kk % nb].wait()
            return carry

        lax.fori_loop(0, NCH_1 // GRP, outer, 0)
        plsc.subcore_barrier()
        pltpu.sync_copy(acc.at[pl.ds(s * RPT, RPT)], out.at[c, pl.ds(s * RPT, RPT)])

    return k


_prop_d1 = _make_prop_d1()


def _pad_tab(tab_ref, u, dh):
    tab_ref[pl.ds(0, N)] = u[:, :dh]
    tab_ref[pl.ds(NP, N)] = u[:, dh:]
    zpad = jnp.zeros((NP - N, dh), jnp.float32)
    tab_ref[pl.ds(N, NP - N)] = zpad
    tab_ref[pl.ds(NP + N, NP - N)] = zpad


def _merge(s_ref):
    v = s_ref[...]
    return jnp.concatenate([v[0, :N], v[1, :N]], axis=1)


def _t1(x_ref, w_ref, cnt_ref, dis_ref, tab_ref):
    deg = 1.0 + cnt_ref[0] + cnt_ref[1]
    dis = lax.rsqrt(deg)
    dis_ref[...] = dis
    z = jnp.dot(x_ref[...], w_ref[...], preferred_element_type=jnp.float32)
    u = z * dis[:N]
    _pad_tab(tab_ref, u, 32)


def _t2(s_ref, dis_ref, b_ref, w_ref, tab_ref):
    dis = dis_ref[...]
    h = jnp.maximum(dis[:N] * _merge(s_ref) + b_ref[...], 0.0)
    u = dis[:N] * jnp.dot(h, w_ref[...], preferred_element_type=jnp.float32)
    _pad_tab(tab_ref, u, 32)


def _t3(s_ref, dis_ref, b_ref, tab_ref):
    dis = dis_ref[...]
    p = dis[:N] * _merge(s_ref) + b_ref[...]
    h = jnp.where(p > 0, p, 0.1 * p)
    _pad_tab(tab_ref, dis[:N] * h, 32)


def _t4(s_ref, dis_ref, ba_ref, wa_ref, wb_ref, tab_ref):
    dis = dis_ref[...]
    p = dis[:N] * _merge(s_ref)
    h = jnp.maximum(jnp.dot(p, wa_ref[...], preferred_element_type=jnp.float32)
                    + ba_ref[...], 0.0)
    u = dis[:N] * jnp.dot(h, wb_ref[...], preferred_element_type=jnp.float32)
    _pad_tab(tab_ref, u, 128)


def _t5(s_ref, dis_ref, b_ref, tab_ref):
    dis = dis_ref[...]
    p = dis[:N] * _merge(s_ref) + b_ref[...]
    h = jnp.where(p > 0, p, 0.1 * p)
    _pad_tab(tab_ref, dis[:N] * h, 128)


def _t6(s_ref, dis_ref, ba_ref, wa_ref, wb_ref, tab_ref):
    dis = dis_ref[...]
    p = dis[:N] * _merge(s_ref)
    h = jnp.maximum(jnp.dot(p, wa_ref[...], preferred_element_type=jnp.float32)
                    + ba_ref[...], 0.0)
    z = jnp.sum(h * wb_ref[...], axis=1, keepdims=True)
    tab_ref[pl.ds(0, N)] = dis[:N] * z
    tab_ref[pl.ds(N, NP - N)] = jnp.zeros((NP - N, 1), jnp.float32)


def _t7(t6_ref, u6_ref, dis_ref, b_ref, out_ref):
    v = t6_ref[...]
    out_ref[...] = (dis_ref[pl.ds(0, N)]
                    * (u6_ref[pl.ds(0, N)] + v[0, :N] + v[1, :N]) + b_ref[...])


def _tc(body, out_shape, *args):
    return pl.pallas_call(body, out_shape=out_shape)(*args)


def kernel(x, edge_index, W0a, b0a, W0b, b0b, W1a, b1a, W1b, b1b, W2a, b2a, W2b, b2b):
    f32 = jnp.float32
    src = edge_index[0]
    dst = edge_index[1]
    srcp = jnp.concatenate([src, jnp.full((EP - E,), N, jnp.int32)])
    dstp = jnp.concatenate([dst, jnp.full((EP - E,), N, jnp.int32)])
    offs = jnp.array([0, NP], jnp.int32)
    srcs_w = (srcp[None, :] + offs[:, None]).reshape(2, 16, NCH_W, CB)
    dst_w = dstp.reshape(16, NCH_W, CB)
    src_1 = srcp.reshape(32, NCH_1, CB)
    dst_1 = dstp.reshape(32, NCH_1, CB)

    b0a_ = b0a.reshape(1, -1)
    b0b_ = b0b.reshape(1, -1)
    b1a_ = b1a.reshape(1, -1)
    b1b_ = b1b.reshape(1, -1)
    b2a_ = b2a.reshape(1, -1)
    b2b_ = b2b.reshape(1, -1)
    w2b_row = W2b.reshape(1, -1)

    ones_tab = jnp.zeros((NP,), f32).at[:N].set(1.0)
    cnt = _prop_d1(ones_tab, src_1, dst_1)
    cnt3 = cnt.reshape(2, NP, 1)

    sd = jax.ShapeDtypeStruct
    dis, tab1 = _tc(_t1, [sd((NP, 1), f32), sd((2 * NP, 32), f32)],
                    x, W0a, cnt3)
    s1 = _prop32(tab1, srcs_w, dst_w)
    tab2 = _tc(_t2, sd((2 * NP, 32), f32), s1, dis, b0a_, W0b)
    s2 = _prop32(tab2, srcs_w, dst_w)
    tab3 = _tc(_t3, sd((2 * NP, 32), f32), s2, dis, b0b_)
    s3 = _prop32(tab3, srcs_w, dst_w)
    tab4 = _tc(_t4, sd((2 * NP, 128), f32), s3, dis, b1a_, W1a, W1b)
    s4 = _prop128(tab4, srcs_w, dst_w)
    tab5 = _tc(_t5, sd((2 * NP, 128), f32), s4, dis, b1b_)
    s5 = _prop128(tab5, srcs_w, dst_w)
    tab6 = _tc(_t6, sd((NP, 1), f32), s5, dis, b2a_, W2a, w2b_row)
    t6 = _prop_d1(tab6.reshape(NP), src_1, dst_1)
    out = _tc(_t7, sd((N, 1), f32), t6.reshape(2, NP, 1), tab6, dis, b2b_)
    return out


# speedup vs baseline: 11.4462x; 1.0219x over previous
import functools

import jax
import jax.numpy as jnp
from jax import lax
from jax.experimental import pallas as pl
from jax.experimental.pallas import tpu as pltpu
from jax.experimental.pallas import tpu_sc as plsc

N = 10000
E = 320000
NP = 10240
RPT = NP // 16
EP = 327680
EPT = EP // 16


def _make_prop_wide(dh, nb, cb, grp):
    nch = EPT // cb
    mesh = plsc.VectorSubcoreMesh(core_axis_name="c", subcore_axis_name="s")

    @functools.partial(
        pl.kernel,
        mesh=mesh,
        out_type=jax.ShapeDtypeStruct((2, NP, dh), jnp.float32),
        compiler_params=pltpu.CompilerParams(use_tc_tiling_on_sc=False),
        scratch_types=[
            pltpu.VMEM_SHARED((NP, dh), jnp.float32),
            pltpu.VMEM((grp, cb), jnp.int32),
            pltpu.VMEM((grp, cb), jnp.int32),
            pltpu.VMEM((nb, cb, dh), jnp.float32),
            pltpu.SemaphoreType.DMA((nb,)),
            pltpu.SemaphoreType.DMA((nb,)),
        ],
    )
    def k(tab, srcs, dsts, out, acc, src_m, dst_m, bufs, gsem, ssem):
        c = lax.axis_index("c")
        s = lax.axis_index("s")
        r0 = s * RPT
        pltpu.sync_copy(tab.at[pl.ds(c * NP + r0, RPT)], acc.at[pl.ds(r0, RPT)])
        plsc.subcore_barrier()

        def outer(g, carry):
            pltpu.sync_copy(srcs.at[c, s, pl.ds(g * grp, grp)], src_m)
            pltpu.sync_copy(dsts.at[s, pl.ds(g * grp, grp)], dst_m)
            gh = [None] * nb
            sh = [None] * nb
            for b in range(nb):
                gh[b] = pltpu.async_copy(tab.at[src_m.at[b]], bufs.at[b],
                                         gsem.at[b])
            for kk in range(grp):
                b = kk % nb
                gh[b].wait()
                sh[b] = pltpu.async_copy(bufs.at[b], acc.at[dst_m.at[kk]],
                                         ssem.at[b], add=True)
                nk = kk + nb
                if nk < grp:
                    sh[b].wait()
                    gh[b] = pltpu.async_copy(tab.at[src_m.at[nk]], bufs.at[b],
                                             gsem.at[b])
            for kk in range(grp - nb, grp):
                sh[kk % nb].wait()
            return carry

        lax.fori_loop(0, nch // grp, outer, 0)
        plsc.subcore_barrier()
        pltpu.sync_copy(acc.at[pl.ds(r0, RPT)], out.at[c, pl.ds(r0, RPT)])

    return k


_prop32 = _make_prop_wide(32, 6, 128, 16)
_prop128 = _make_prop_wide(128, 4, 64, 32)


def _make_prop_d1():
    mesh = plsc.VectorSubcoreMesh(core_axis_name="c", subcore_axis_name="s")
    nstep = EP // 32 // 16

    @functools.partial(
        pl.kernel,
        mesh=mesh,
        out_type=jax.ShapeDtypeStruct((2, NP), jnp.float32),
        compiler_params=pltpu.CompilerParams(use_tc_tiling_on_sc=False,
                                             needs_layout_passes=False),
        scratch_types=[
            pltpu.VMEM_SHARED((16, NP), jnp.float32),
            pltpu.VMEM((NP,), jnp.float32),
            pltpu.VMEM((NP,), jnp.float32),
            pltpu.VMEM((nstep, 16), jnp.int32),
            pltpu.VMEM((nstep, 16), jnp.int32),
            pltpu.VMEM((16, RPT), jnp.float32),
            pltpu.VMEM((RPT,), jnp.float32),
        ],
    )
    def k(tab, srcs, dsts, out, part, tabv, loc, src_m, dst_m, tmp, res):
        c = lax.axis_index("c")
        s = lax.axis_index("s")
        w = 2 * s + c
        pltpu.sync_copy(tab, tabv)
        pltpu.sync_copy(srcs.at[w], src_m)
        pltpu.sync_copy(dsts.at[w], dst_m)

        def z(j, carry):
            loc[pl.ds(j * 16, 16)] = jnp.zeros((16,), jnp.float32)
            return carry

        lax.fori_loop(0, NP // 16, z, 0)

        def step(j, carry):
            g = plsc.load_gather(tabv, [src_m[j]])
            plsc.addupdate_scatter(loc, [dst_m[j]], g)
            return carry

        lax.fori_loop(0, nstep, step, 0)
        pltpu.sync_copy(loc, part.at[s])
        plsc.subcore_barrier()

        r0 = s * RPT
        for t in range(16):
            pltpu.sync_copy(part.at[t, pl.ds(r0, RPT)], tmp.at[t])

        def red(j, carry):
            v = tmp[0, pl.ds(j * 16, 16)]
            for t in range(1, 16):
                v = v + tmp[t, pl.ds(j * 16, 16)]
            res[pl.ds(j * 16, 16)] = v
            return carry

        lax.fori_loop(0, RPT // 16, red, 0)
        pltpu.sync_copy(res, out.at[c, pl.ds(r0, RPT)])

    return k


_prop_d1 = _make_prop_d1()


def _pad_tab(tab_ref, u, dh):
    tab_ref[pl.ds(0, N)] = u[:, :dh]
    tab_ref[pl.ds(NP, N)] = u[:, dh:]
    zpad = jnp.zeros((NP - N, dh), jnp.float32)
    tab_ref[pl.ds(N, NP - N)] = zpad
    tab_ref[pl.ds(NP + N, NP - N)] = zpad


def _merge(s_ref):
    v = s_ref[...]
    return jnp.concatenate([v[0, :N], v[1, :N]], axis=1)


def _t1(x_ref, w_ref, cnt_ref, dis_ref, tab_ref):
    deg = 1.0 + cnt_ref[0] + cnt_ref[1]
    dis = lax.rsqrt(deg)
    dis_ref[...] = dis
    z = jnp.dot(x_ref[...], w_ref[...], preferred_element_type=jnp.float32)
    u = z * dis[:N]
    _pad_tab(tab_ref, u, 32)


def _t2(s_ref, dis_ref, b_ref, w_ref, tab_ref):
    dis = dis_ref[...]
    h = jnp.maximum(dis[:N] * _merge(s_ref) + b_ref[...], 0.0)
    u = dis[:N] * jnp.dot(h, w_ref[...], preferred_element_type=jnp.float32)
    _pad_tab(tab_ref, u, 32)


def _t3(s_ref, dis_ref, b_ref, tab_ref):
    dis = dis_ref[...]
    p = dis[:N] * _merge(s_ref) + b_ref[...]
    h = jnp.where(p > 0, p, 0.1 * p)
    _pad_tab(tab_ref, dis[:N] * h, 32)


def _t4(s_ref, dis_ref, ba_ref, wa_ref, wb_ref, tab_ref):
    dis = dis_ref[...]
    p = dis[:N] * _merge(s_ref)
    h = jnp.maximum(jnp.dot(p, wa_ref[...], preferred_element_type=jnp.float32)
                    + ba_ref[...], 0.0)
    u = dis[:N] * jnp.dot(h, wb_ref[...], preferred_element_type=jnp.float32)
    _pad_tab(tab_ref, u, 128)


def _t5(s_ref, dis_ref, b_ref, tab_ref):
    dis = dis_ref[...]
    p = dis[:N] * _merge(s_ref) + b_ref[...]
    h = jnp.where(p > 0, p, 0.1 * p)
    _pad_tab(tab_ref, dis[:N] * h, 128)


def _t6(s_ref, dis_ref, ba_ref, wa_ref, wb_ref, tab_ref):
    dis = dis_ref[...]
    p = dis[:N] * _merge(s_ref)
    h = jnp.maximum(jnp.dot(p, wa_ref[...], preferred_element_type=jnp.float32)
                    + ba_ref[...], 0.0)
    z = jnp.sum(h * wb_ref[...], axis=1, keepdims=True)
    tab_ref[pl.ds(0, N)] = dis[:N] * z
    tab_ref[pl.ds(N, NP - N)] = jnp.zeros((NP - N, 1), jnp.float32)


def _t7(t6_ref, u6_ref, dis_ref, b_ref, out_ref):
    v = t6_ref[...]
    out_ref[...] = (dis_ref[pl.ds(0, N)]
                    * (u6_ref[pl.ds(0, N)] + v[0, :N] + v[1, :N]) + b_ref[...])


def _tc(body, out_shape, *args):
    return pl.pallas_call(body, out_shape=out_shape)(*args)


def kernel(x, edge_index, W0a, b0a, W0b, b0b, W1a, b1a, W1b, b1b, W2a, b2a, W2b, b2b):
    f32 = jnp.float32
    src = edge_index[0]
    dst = edge_index[1]
    srcp = jnp.concatenate([src, jnp.full((EP - E,), N, jnp.int32)])
    dstp = jnp.concatenate([dst, jnp.full((EP - E,), N, jnp.int32)])
    offs = jnp.array([0, NP], jnp.int32)
    srcs_sh = srcp[None, :] + offs[:, None]
    srcs_w32 = srcs_sh.reshape(2, 16, EPT // 128, 128)
    dst_w32 = dstp.reshape(16, EPT // 128, 128)
    srcs_w128 = srcs_sh.reshape(2, 16, EPT // 64, 64)
    dst_w128 = dstp.reshape(16, EPT // 64, 64)
    src_1 = srcp.reshape(32, EP // 32 // 16, 16)
    dst_1 = dstp.reshape(32, EP // 32 // 16, 16)

    b0a_ = b0a.reshape(1, -1)
    b0b_ = b0b.reshape(1, -1)
    b1a_ = b1a.reshape(1, -1)
    b1b_ = b1b.reshape(1, -1)
    b2a_ = b2a.reshape(1, -1)
    b2b_ = b2b.reshape(1, -1)
    w2b_row = W2b.reshape(1, -1)

    ones_tab = jnp.zeros((NP,), f32).at[:N].set(1.0)
    cnt = _prop_d1(ones_tab, src_1, dst_1)
    cnt3 = cnt.reshape(2, NP, 1)

    sd = jax.ShapeDtypeStruct
    dis, tab1 = _tc(_t1, [sd((NP, 1), f32), sd((2 * NP, 32), f32)],
                    x, W0a, cnt3)
    s1 = _prop32(tab1, srcs_w32, dst_w32)
    tab2 = _tc(_t2, sd((2 * NP, 32), f32), s1, dis, b0a_, W0b)
    s2 = _prop32(tab2, srcs_w32, dst_w32)
    tab3 = _tc(_t3, sd((2 * NP, 32), f32), s2, dis, b0b_)
    s3 = _prop32(tab3, srcs_w32, dst_w32)
    tab4 = _tc(_t4, sd((2 * NP, 128), f32), s3, dis, b1a_, W1a, W1b)
    s4 = _prop128(tab4, srcs_w128, dst_w128)
    tab5 = _tc(_t5, sd((2 * NP, 128), f32), s4, dis, b1b_)
    s5 = _prop128(tab5, srcs_w128, dst_w128)
    tab6 = _tc(_t6, sd((NP, 1), f32), s5, dis, b2a_, W2a, w2b_row)
    t6 = _prop_d1(tab6.reshape(NP), src_1, dst_1)
    out = _tc(_t7, sd((N, 1), f32), t6.reshape(2, NP, 1), tab6, dis, b2b_)
    return out
